# Initial kernel scaffold; baseline (speedup 1.0000x reference)
import jax
import jax.numpy as jnp
from jax.experimental import pallas as pl


def kernel(user, positive, negative, user_emb, item_emb, edge_index, edge_vals, W_gcn_0, b_gcn_0, W_mlp_0, b_mlp_0, W_gcn_1, b_gcn_1, W_mlp_1, b_mlp_1):
    raise NotImplementedError("write your pallas kernel here")


# speedup vs baseline: 10.5305x; 10.5305x over previous
import functools

import jax
import jax.numpy as jnp
from jax import lax
from jax.experimental import pallas as pl
from jax.experimental.pallas import tpu as pltpu
from jax.experimental.pallas import tpu_sc as plsc

NU = 50000
NI = 50000
NN = NU + NI
EE = 1600000
DD = 32
HH = 16
BB = 4096
REG_L = 0.0001

NC = 2
NS = 16
SUB = 8
KCH = SUB * 128
CHUNKS = 98
EPT = CHUNKS * KCH
EP = EPT * NS
RPT = EPT // 128
NP = 100096
ROWS_PT = NP // NS
ZR = 136
GIDX = 3 * BB
GPT = GIDX // NS

_MESH = plsc.VectorSubcoreMesh(
    core_axis_name="c", subcore_axis_name="s", num_cores=NC, num_subcores=NS)


_GDN = lax.GatherDimensionNumbers(
    offset_dims=(), collapsed_slice_dims=(0,), start_index_map=(0,))


def _splat(v16, i):
    return lax.gather(
        v16, jnp.full((16, 1), i, jnp.int32), _GDN, (1,),
        mode=lax.GatherScatterMode.PROMISE_IN_BOUNDS)


def _spmm_body(ego_tab, src2d, dst2d, val2d, out,
               srcv, dstv, valv, rows, zbuf, acc, sem, gsem):
    c = lax.axis_index("c")
    s = lax.axis_index("s")

    def _zb(i, carry):
        zbuf[i, :] = jnp.zeros((HH,), jnp.float32)
        return carry
    lax.fori_loop(0, ZR, _zb, 0)

    def _za(i, carry):
        pltpu.sync_copy(zbuf, acc.at[pl.ds(s * ROWS_PT + i * ZR, ZR)])
        return carry
    lax.fori_loop(0, ROWS_PT // ZR, _za, 0)
    plsc.subcore_barrier()

    def _chunk(k, carry):
        row0 = s * RPT + k * SUB
        cp1 = pltpu.async_copy(src2d.at[pl.ds(row0, SUB)], srcv, sem)
        cp2 = pltpu.async_copy(dst2d.at[pl.ds(row0, SUB)], dstv, sem)
        cp3 = pltpu.async_copy(val2d.at[pl.ds(row0, SUB)], valv, sem)
        cp1.wait()
        cp2.wait()
        cp3.wait()
        for j in range(SUB):
            pltpu.async_copy(ego_tab.at[c].at[srcv.at[j]], rows.at[j], gsem)
        for j in range(SUB):
            pltpu.make_async_copy(
                ego_tab.at[c].at[srcv.at[j]], rows.at[j], gsem).wait()
        for j in range(SUB):
            def _grp(g, carry, jj=j):
                base = g * 16
                v16 = valv[jj, pl.ds(base, 16)]
                for i in range(16):
                    r = rows[jj, base + i, :]
                    rows[jj, base + i, :] = r * _splat(v16, i)
                return carry
            lax.fori_loop(0, 8, _grp, 0)
        for j in range(SUB):
            pltpu.sync_copy(rows.at[j], acc.at[dstv.at[j]], add=True)
        return carry
    lax.fori_loop(0, CHUNKS, _chunk, 0)

    plsc.subcore_barrier()
    pltpu.sync_copy(acc.at[pl.ds(s * ROWS_PT, ROWS_PT)],
                    out.at[c].at[pl.ds(s * ROWS_PT, ROWS_PT)])


_spmm = pl.kernel(
    _spmm_body,
    out_type=jax.ShapeDtypeStruct((NC, NP, HH), jnp.float32),
    mesh=_MESH,
    compiler_params=pltpu.CompilerParams(use_tc_tiling_on_sc=False),
    scratch_types=[
        pltpu.VMEM((SUB, 128), jnp.int32),
        pltpu.VMEM((SUB, 128), jnp.int32),
        pltpu.VMEM((SUB, 128), jnp.float32),
        pltpu.VMEM((SUB, 128, HH), jnp.float32),
        pltpu.VMEM((ZR, HH), jnp.float32),
        pltpu.VMEM_SHARED((NP, HH), jnp.float32),
        pltpu.SemaphoreType.DMA,
        pltpu.SemaphoreType.DMA,
    ],
)


def _gather3_body(t0, t1, t2, idx2d, out, idxv, rows, sem, gsem):
    c = lax.axis_index("c")
    s = lax.axis_index("s")
    pltpu.async_copy(idx2d.at[pl.ds(s * 6, 6)], idxv, sem).wait()
    for t, tab in enumerate((t0, t1, t2)):
        for j in range(6):
            pltpu.async_copy(tab.at[c].at[idxv.at[j]], rows.at[j], gsem)
        for j in range(6):
            pltpu.make_async_copy(
                tab.at[c].at[idxv.at[j]], rows.at[j], gsem).wait()
        for j in range(6):
            pltpu.sync_copy(
                rows.at[j],
                out.at[t].at[c].at[pl.ds(s * GPT + j * 128, 128)])


_gather3 = pl.kernel(
    _gather3_body,
    out_type=jax.ShapeDtypeStruct((3, NC, GIDX, HH), jnp.float32),
    mesh=_MESH,
    compiler_params=pltpu.CompilerParams(use_tc_tiling_on_sc=False),
    scratch_types=[
        pltpu.VMEM((6, 128), jnp.int32),
        pltpu.VMEM((6, 128, HH), jnp.float32),
        pltpu.SemaphoreType.DMA,
        pltpu.SemaphoreType.DMA,
    ],
)


def _dense_body(x_ref, wg_ref, bg_ref, wm_ref, bm_ref, o_ref):
    xx = jnp.concatenate([x_ref[0], x_ref[1]], axis=1)
    h = jnp.dot(xx, wg_ref[...], precision=lax.Precision.HIGHEST) + bg_ref[...]
    h = jnp.where(h >= 0, h, 0.2 * h)
    e = jnp.dot(h, wm_ref[...], precision=lax.Precision.HIGHEST) + bm_ref[...]
    o_ref[0] = e[:, :HH]
    o_ref[1] = e[:, HH:]


_DENSE_BR = 3128
_dense = pl.pallas_call(
    _dense_body,
    grid=(NP // _DENSE_BR,),
    in_specs=[
        pl.BlockSpec((NC, _DENSE_BR, HH), lambda i: (0, i, 0)),
        pl.BlockSpec((DD, DD), lambda i: (0, 0)),
        pl.BlockSpec((1, DD), lambda i: (0, 0)),
        pl.BlockSpec((DD, DD), lambda i: (0, 0)),
        pl.BlockSpec((1, DD), lambda i: (0, 0)),
    ],
    out_specs=pl.BlockSpec((NC, _DENSE_BR, HH), lambda i: (0, i, 0)),
    out_shape=jax.ShapeDtypeStruct((NC, NP, HH), jnp.float32),
)


def _rownorm(x):
    nr = jnp.sqrt(jnp.sum(x * x, axis=1, keepdims=True))
    return x / jnp.maximum(nr, 1e-12)


def _final_body(g_ref, wg_ref, bg_ref, wm_ref, bm_ref, bpr_ref, reg_ref):
    i = pl.program_id(0)
    def rows32(t, k):
        return jnp.concatenate([g_ref[t, 0, k], g_ref[t, 1, k]], axis=1)
    u0, p0, n0 = rows32(0, 0), rows32(0, 1), rows32(0, 2)
    u1, p1, n1 = rows32(1, 0), rows32(1, 1), rows32(1, 2)
    s2 = jnp.concatenate([rows32(2, 0), rows32(2, 1), rows32(2, 2)], axis=0)
    h = jnp.dot(s2, wg_ref[...], precision=lax.Precision.HIGHEST) + bg_ref[...]
    h = jnp.where(h >= 0, h, 0.2 * h)
    e2 = jnp.dot(h, wm_ref[...], precision=lax.Precision.HIGHEST) + bm_ref[...]
    u1, p1, n1 = _rownorm(u1), _rownorm(p1), _rownorm(n1)
    u2 = _rownorm(e2[:_FB])
    p2 = _rownorm(e2[_FB:2 * _FB])
    n2 = _rownorm(e2[2 * _FB:])
    pos = (jnp.sum(u0 * p0, axis=1) + jnp.sum(u1 * p1, axis=1)
           + jnp.sum(u2 * p2, axis=1))
    neg = (jnp.sum(u0 * n0, axis=1) + jnp.sum(u1 * n1, axis=1)
           + jnp.sum(u2 * n2, axis=1))
    d = neg - pos
    sp_sum = jnp.sum(jnp.maximum(d, 0.0) + jnp.log1p(jnp.exp(-jnp.abs(d))))
    sq_sum = jnp.sum(u0 * u0) + jnp.sum(p0 * p0) + jnp.sum(n0 * n0)

    @pl.when(i == 0)
    def _init():
        bpr_ref[...] = jnp.zeros((1, 1), jnp.float32)
        reg_ref[...] = jnp.zeros((1, 1), jnp.float32)

    bpr_ref[...] += jnp.reshape(sp_sum, (1, 1))
    reg_ref[...] += jnp.reshape(sq_sum, (1, 1))


_FB = 256
_final = pl.pallas_call(
    _final_body,
    grid=(BB // _FB,),
    in_specs=[
        pl.BlockSpec((3, NC, 3, _FB, HH), lambda i: (0, 0, 0, i, 0)),
        pl.BlockSpec((DD, DD), lambda i: (0, 0)),
        pl.BlockSpec((1, DD), lambda i: (0, 0)),
        pl.BlockSpec((DD, DD), lambda i: (0, 0)),
        pl.BlockSpec((1, DD), lambda i: (0, 0)),
    ],
    out_specs=(pl.BlockSpec((1, 1), lambda i: (0, 0)),
               pl.BlockSpec((1, 1), lambda i: (0, 0))),
    out_shape=(jax.ShapeDtypeStruct((1, 1), jnp.float32),
               jax.ShapeDtypeStruct((1, 1), jnp.float32)),
)


def kernel(user, positive, negative, user_emb, item_emb, edge_index, edge_vals,
           W_gcn_0, b_gcn_0, W_mlp_0, b_mlp_0, W_gcn_1, b_gcn_1, W_mlp_1,
           b_mlp_1):
    ego0 = jnp.concatenate([user_emb, item_emb], axis=0)
    ego0_tab = ego0.reshape(NN, NC, HH).transpose(1, 0, 2)
    ego0_tab = jnp.concatenate(
        [ego0_tab, jnp.zeros((NC, NP - NN, HH), jnp.float32)], axis=1)

    pad = EP - EE
    src = jnp.concatenate([edge_index[0], jnp.zeros((pad,), jnp.int32)])
    dst = jnp.concatenate([edge_index[1], jnp.zeros((pad,), jnp.int32)])
    vals = jnp.concatenate([edge_vals, jnp.zeros((pad,), jnp.float32)])
    src2d = src.reshape(-1, 128)
    dst2d = dst.reshape(-1, 128)
    val2d = vals.reshape(-1, 128)

    side1_tab = _spmm(ego0_tab, src2d, dst2d, val2d)
    ego1_tab = _dense(side1_tab, W_gcn_0, b_gcn_0, W_mlp_0, b_mlp_0)
    side2_tab = _spmm(ego1_tab, src2d, dst2d, val2d)

    idx_all = jnp.concatenate([user, NU + positive, NU + negative])
    idx2d = idx_all.reshape(-1, 128)
    gath = _gather3(ego0_tab, ego1_tab, side2_tab, idx2d)

    gath5 = gath.reshape(3, NC, 3, BB, HH)
    sp_sum, sq_sum = _final(gath5, W_gcn_1, b_gcn_1, W_mlp_1, b_mlp_1)
    bpr = sp_sum[0, 0] / BB
    reg = REG_L * 0.5 * sq_sum[0, 0] / BB
    return (bpr, reg)


# speedup vs baseline: 15.7808x; 1.4986x over previous
import functools

import jax
import jax.numpy as jnp
from jax import lax
from jax.experimental import pallas as pl
from jax.experimental.pallas import tpu as pltpu
from jax.experimental.pallas import tpu_sc as plsc

NU = 50000
NI = 50000
NN = NU + NI
EE = 1600000
DD = 32
HH = 16
BB = 4096
REG_L = 0.0001

NC = 2
NS = 16
SUB = 6
RT = EE // 128
RPT = 781
FCH = 130
TAIL = RPT - FCH * SUB
XTRA = RT - RPT * NS
NP = 100096
ROWS_PT = NP // NS
GIDX = 3 * BB
GPT = GIDX // NS

_MESH = plsc.VectorSubcoreMesh(
    core_axis_name="c", subcore_axis_name="s", num_cores=NC, num_subcores=NS)

_GDN = lax.GatherDimensionNumbers(
    offset_dims=(), collapsed_slice_dims=(0,), start_index_map=(0,))


def _splat(v16, i):
    return lax.gather(
        v16, jnp.full((16, 1), i, jnp.int32), _GDN, (1,),
        mode=lax.GatherScatterMode.PROMISE_IN_BOUNDS)


def _spmm_body(tab, e3, v2, out, bufs, acc, sems):
    c = lax.axis_index("c")
    s = lax.axis_index("s")
    lo = s * RPT

    sets = (bufs[0] + (sems[0], sems[1]), bufs[1] + (sems[2], sems[3]))

    def issue_idx(k, p, n=SUB):
        sv, dv, vv, rw, ise, gse = sets[p]
        row0 = lo + k * SUB
        pltpu.async_copy(e3.at[0].at[pl.ds(row0, n)], sv.at[pl.ds(0, n)], ise)
        pltpu.async_copy(e3.at[1].at[pl.ds(row0, n)], dv.at[pl.ds(0, n)], ise)
        pltpu.async_copy(v2.at[pl.ds(row0, n)], vv.at[pl.ds(0, n)], ise)

    def wait_idx(k, p, n=SUB):
        sv, dv, vv, rw, ise, gse = sets[p]
        row0 = lo + k * SUB
        pltpu.make_async_copy(
            e3.at[0].at[pl.ds(row0, n)], sv.at[pl.ds(0, n)], ise).wait()
        pltpu.make_async_copy(
            e3.at[1].at[pl.ds(row0, n)], dv.at[pl.ds(0, n)], ise).wait()
        pltpu.make_async_copy(
            v2.at[pl.ds(row0, n)], vv.at[pl.ds(0, n)], ise).wait()

    def fire(p, n=SUB):
        sv, dv, vv, rw, ise, gse = sets[p]

        def _tr(g, carry):
            jj = g // 8
            base = (g % 8) * 16
            sv[jj, pl.ds(base, 16)] = sv[jj, pl.ds(base, 16)] * 2 + c
            return carry
        lax.fori_loop(0, n * 8, _tr, 0)
        for j in range(n):
            pltpu.async_copy(tab.at[sv.at[j]], rw.at[j], gse)

    def drain(p, n=SUB):
        sv, dv, vv, rw, ise, gse = sets[p]
        for j in range(n):
            pltpu.make_async_copy(tab.at[sv.at[j]], rw.at[j], gse).wait()

    def scale(p, n=SUB):
        sv, dv, vv, rw, ise, gse = sets[p]

        def _grp(g, carry):
            jj = g // 8
            base = (g % 8) * 16
            v16 = vv[jj, pl.ds(base, 16)]
            for i in range(16):
                r = rw[jj, base + i, :]
                rw[jj, base + i, :] = r * _splat(v16, i)
            return carry
        lax.fori_loop(0, n * 8, _grp, 0)

    def scatter(p, n=SUB):
        sv, dv, vv, rw, ise, gse = sets[p]
        for j in range(n):
            pltpu.sync_copy(rw.at[j], acc.at[dv.at[j]], add=True)

    rw0 = sets[0][3]

    def _zb(g, carry):
        rw0[g // 128, g % 128, :] = jnp.zeros((HH,), jnp.float32)
        return carry
    lax.fori_loop(0, SUB * 128, _zb, 0)

    def _za(i, carry):
        pltpu.sync_copy(rw0.at[i % SUB],
                        acc.at[pl.ds(s * ROWS_PT + i * 128, 128)])
        return carry
    lax.fori_loop(0, ROWS_PT // 128, _za, 0)
    pltpu.sync_copy(rw0.at[0].at[pl.ds(0, ROWS_PT - 48 * 128)],
                    acc.at[pl.ds(s * ROWS_PT + 48 * 128,
                                 ROWS_PT - 48 * 128)])
    plsc.subcore_barrier()

    issue_idx(0, 0)
    issue_idx(1, 1)
    wait_idx(0, 0)
    fire(0)

    def _pair(i, carry):
        for (off, p, pn) in ((0, 0, 1), (1, 1, 0)):
            k = 2 * i + off
            drain(p)
            scale(p)
            scatter(p)
            issue_idx(k + 2, p)
            wait_idx(k + 1, pn)
            fire(pn)
        return carry
    lax.fori_loop(0, FCH // 2 - 1, _pair, 0)
    drain(0)
    scale(0)
    scatter(0)
    wait_idx(FCH - 1, 1)
    fire(1)
    drain(1)
    scale(1)
    scatter(1)
    issue_idx(FCH, 0, TAIL)
    wait_idx(FCH, 0, TAIL)
    fire(0, TAIL)
    drain(0, TAIL)
    scale(0, TAIL)
    scatter(0, TAIL)

    @pl.when(s == NS - 1)
    def _extra():
        sv, dv, vv, rw, ise, gse = sets[1]
        row0 = NS * RPT
        pltpu.async_copy(e3.at[0].at[pl.ds(row0, XTRA)],
                         sv.at[pl.ds(0, XTRA)], ise)
        pltpu.async_copy(e3.at[1].at[pl.ds(row0, XTRA)],
                         dv.at[pl.ds(0, XTRA)], ise)
        pltpu.async_copy(v2.at[pl.ds(row0, XTRA)],
                         vv.at[pl.ds(0, XTRA)], ise)
        pltpu.make_async_copy(e3.at[0].at[pl.ds(row0, XTRA)],
                              sv.at[pl.ds(0, XTRA)], ise).wait()
        pltpu.make_async_copy(e3.at[1].at[pl.ds(row0, XTRA)],
                              dv.at[pl.ds(0, XTRA)], ise).wait()
        pltpu.make_async_copy(v2.at[pl.ds(row0, XTRA)],
                              vv.at[pl.ds(0, XTRA)], ise).wait()
        fire(1, XTRA)
        drain(1, XTRA)
        scale(1, XTRA)
        scatter(1, XTRA)

    plsc.subcore_barrier()
    pltpu.sync_copy(acc.at[pl.ds(s * ROWS_PT, ROWS_PT)],
                    out.at[c].at[pl.ds(s * ROWS_PT, ROWS_PT)])


def _make_spmm():
    def body(tab, e3, v2, out, sv, dv, vv, rw, sv2, dv2, vv2, rw2,
             acc, isem, gsem, isem2, gsem2):
        _spmm_body(tab, e3, v2, out,
                   ((sv, dv, vv, rw), (sv2, dv2, vv2, rw2)),
                   acc, (isem, gsem, isem2, gsem2))

    iset = [pltpu.VMEM((SUB, 128), jnp.int32),
            pltpu.VMEM((SUB, 128), jnp.int32),
            pltpu.VMEM((SUB, 128), jnp.float32),
            pltpu.VMEM((SUB, 128, HH), jnp.float32)]
    return pl.kernel(
        body,
        out_type=jax.ShapeDtypeStruct((NC, NP, HH), jnp.float32),
        mesh=_MESH,
        compiler_params=pltpu.CompilerParams(use_tc_tiling_on_sc=False),
        scratch_types=iset + iset + [
            pltpu.VMEM_SHARED((NP, HH), jnp.float32),
            pltpu.SemaphoreType.DMA,
            pltpu.SemaphoreType.DMA,
            pltpu.SemaphoreType.DMA,
            pltpu.SemaphoreType.DMA,
        ],
    )


_spmm = _make_spmm()


def _gather3_body(t0, t1, t2, idx2d, out, idxv, gidx, rows, sem, gsem):
    c = lax.axis_index("c")
    s = lax.axis_index("s")
    pltpu.async_copy(idx2d.at[pl.ds(s * 6, 6)], idxv, sem).wait()

    def _tr(g, carry):
        jj = g // 8
        base = (g % 8) * 16
        gidx[jj, pl.ds(base, 16)] = idxv[jj, pl.ds(base, 16)] * 2 + c
        return carry
    lax.fori_loop(0, 48, _tr, 0)
    for t, tab in enumerate((t0, t1)):
        for j in range(6):
            pltpu.async_copy(tab.at[gidx.at[j]], rows.at[j], gsem)
        for j in range(6):
            pltpu.make_async_copy(tab.at[gidx.at[j]], rows.at[j], gsem).wait()
        for j in range(6):
            pltpu.sync_copy(
                rows.at[j],
                out.at[t].at[c].at[pl.ds(s * GPT + j * 128, 128)])
    for j in range(6):
        pltpu.async_copy(t2.at[c].at[idxv.at[j]], rows.at[j], gsem)
    for j in range(6):
        pltpu.make_async_copy(t2.at[c].at[idxv.at[j]], rows.at[j], gsem).wait()
    for j in range(6):
        pltpu.sync_copy(
            rows.at[j],
            out.at[2].at[c].at[pl.ds(s * GPT + j * 128, 128)])


_gather3 = pl.kernel(
    _gather3_body,
    out_type=jax.ShapeDtypeStruct((3, NC, GIDX, HH), jnp.float32),
    mesh=_MESH,
    compiler_params=pltpu.CompilerParams(use_tc_tiling_on_sc=False),
    scratch_types=[
        pltpu.VMEM((6, 128), jnp.int32),
        pltpu.VMEM((6, 128), jnp.int32),
        pltpu.VMEM((6, 128, HH), jnp.float32),
        pltpu.SemaphoreType.DMA,
        pltpu.SemaphoreType.DMA,
    ],
)


def _dense_body(x_ref, bd0_ref, bd1_ref, bdm_ref, bgt_ref, bmt_ref, o_ref):
    x0p = x_ref[0]
    x1p = x_ref[1]
    h = (jnp.dot(x0p, bd0_ref[...], precision=lax.Precision.HIGHEST)
         + jnp.dot(x1p, bd1_ref[...], precision=lax.Precision.HIGHEST)
         + bgt_ref[...])
    h = jnp.where(h >= 0, h, 0.2 * h)
    e = jnp.dot(h, bdm_ref[...], precision=lax.Precision.HIGHEST) + bmt_ref[...]
    o_ref[...] = e.reshape(2 * _BRP, 128)


_BRP = 3128
_dense = pl.pallas_call(
    _dense_body,
    grid=(NP // 8 // _BRP,),
    in_specs=[
        pl.BlockSpec((NC, _BRP, 128), lambda i: (0, i, 0)),
        pl.BlockSpec((128, 256), lambda i: (0, 0)),
        pl.BlockSpec((128, 256), lambda i: (0, 0)),
        pl.BlockSpec((256, 256), lambda i: (0, 0)),
        pl.BlockSpec((1, 256), lambda i: (0, 0)),
        pl.BlockSpec((1, 256), lambda i: (0, 0)),
    ],
    out_specs=pl.BlockSpec((2 * _BRP, 128), lambda i: (i, 0)),
    out_shape=jax.ShapeDtypeStruct((NP // 4, 128), jnp.float32),
)


def _rownorm(x):
    nr = jnp.sqrt(jnp.sum(x * x, axis=1, keepdims=True))
    return x / jnp.maximum(nr, 1e-12)


def _final_body(g_ref, wg_ref, bg_ref, wm_ref, bm_ref, bpr_ref, reg_ref):
    i = pl.program_id(0)
    def rows32(t, k):
        return jnp.concatenate([g_ref[t, 0, k], g_ref[t, 1, k]], axis=1)
    u0, p0, n0 = rows32(0, 0), rows32(0, 1), rows32(0, 2)
    u1, p1, n1 = rows32(1, 0), rows32(1, 1), rows32(1, 2)
    s2 = jnp.concatenate([rows32(2, 0), rows32(2, 1), rows32(2, 2)], axis=0)
    h = jnp.dot(s2, wg_ref[...], precision=lax.Precision.HIGHEST) + bg_ref[...]
    h = jnp.where(h >= 0, h, 0.2 * h)
    e2 = jnp.dot(h, wm_ref[...], precision=lax.Precision.HIGHEST) + bm_ref[...]
    u1, p1, n1 = _rownorm(u1), _rownorm(p1), _rownorm(n1)
    u2 = _rownorm(e2[:_FB])
    p2 = _rownorm(e2[_FB:2 * _FB])
    n2 = _rownorm(e2[2 * _FB:])
    pos = (jnp.sum(u0 * p0, axis=1) + jnp.sum(u1 * p1, axis=1)
           + jnp.sum(u2 * p2, axis=1))
    neg = (jnp.sum(u0 * n0, axis=1) + jnp.sum(u1 * n1, axis=1)
           + jnp.sum(u2 * n2, axis=1))
    d = neg - pos
    sp_sum = jnp.sum(jnp.maximum(d, 0.0) + jnp.log1p(jnp.exp(-jnp.abs(d))))
    sq_sum = jnp.sum(u0 * u0) + jnp.sum(p0 * p0) + jnp.sum(n0 * n0)

    @pl.when(i == 0)
    def _init():
        bpr_ref[...] = jnp.zeros((1, 1), jnp.float32)
        reg_ref[...] = jnp.zeros((1, 1), jnp.float32)

    bpr_ref[...] += jnp.reshape(sp_sum, (1, 1))
    reg_ref[...] += jnp.reshape(sq_sum, (1, 1))


_FB = 256
_final = pl.pallas_call(
    _final_body,
    grid=(BB // _FB,),
    in_specs=[
        pl.BlockSpec((3, NC, 3, _FB, HH), lambda i: (0, 0, 0, i, 0)),
        pl.BlockSpec((DD, DD), lambda i: (0, 0)),
        pl.BlockSpec((1, DD), lambda i: (0, 0)),
        pl.BlockSpec((DD, DD), lambda i: (0, 0)),
        pl.BlockSpec((1, DD), lambda i: (0, 0)),
    ],
    out_specs=(pl.BlockSpec((1, 1), lambda i: (0, 0)),
               pl.BlockSpec((1, 1), lambda i: (0, 0))),
    out_shape=(jax.ShapeDtypeStruct((1, 1), jnp.float32),
               jax.ShapeDtypeStruct((1, 1), jnp.float32)),
)


def kernel(user, positive, negative, user_emb, item_emb, edge_index, edge_vals,
           W_gcn_0, b_gcn_0, W_mlp_0, b_mlp_0, W_gcn_1, b_gcn_1, W_mlp_1,
           b_mlp_1):
    ego0_flat = jnp.concatenate([user_emb, item_emb], axis=0).reshape(
        2 * NN, HH)
    e3 = edge_index.reshape(2, RT, 128)
    v2 = edge_vals.reshape(RT, 128)

    side1_tab = _spmm(ego0_flat, e3, v2)

    ey8 = jnp.eye(8, dtype=jnp.float32)
    bd0 = jnp.kron(ey8, W_gcn_0[:HH, :])
    bd1 = jnp.kron(ey8, W_gcn_0[HH:, :])
    bdm = jnp.kron(ey8, W_mlp_0)
    bgt = jnp.tile(b_gcn_0, (1, 8))
    bmt = jnp.tile(b_mlp_0, (1, 8))

    sidep = side1_tab.reshape(NC, NP // 8, 128)
    ego1_pack = _dense(sidep, bd0, bd1, bdm, bgt, bmt)
    ego1_flat = ego1_pack.reshape(2 * NP, HH)

    side2_tab = _spmm(ego1_flat, e3, v2)

    idx_all = jnp.concatenate([user, NU + positive, NU + negative])
    idx2d = idx_all.reshape(-1, 128)
    gath = _gather3(ego0_flat, ego1_flat, side2_tab, idx2d)

    gath5 = gath.reshape(3, NC, 3, BB, HH)
    sp_sum, sq_sum = _final(gath5, W_gcn_1, b_gcn_1, W_mlp_1, b_mlp_1)
    bpr = sp_sum[0, 0] / BB
    reg = REG_L * 0.5 * sq_sum[0, 0] / BB
    return (bpr, reg)
